# Initial kernel scaffold; baseline (speedup 1.0000x reference)
import jax
import jax.numpy as jnp
from jax.experimental import pallas as pl


def kernel(mean, variance, confidence):
    raise NotImplementedError("write your pallas kernel here")


# speedup vs baseline: 10.5999x; 10.5999x over previous
import jax
import jax.numpy as jnp
from jax.experimental import pallas as pl

_STRIDE = 16
_MIN_CONF = 0.1


def _body(my_ref, vr_ref, mx_ref, vc_ref, c_ref, out_ref):
    hs, np_ = out_ref.shape[1], my_ref.shape[2]
    ws = out_ref.shape[2]
    my = my_ref[0]
    hv_r = 0.5 / vr_ref[0]
    mx = mx_ref[0]
    hv_c = 0.5 / vc_ref[0]
    c = c_ref[0]
    ceff = jnp.where(c > _MIN_CONF, c, 0.0)

    y = jax.lax.broadcasted_iota(jnp.int32, (hs, np_), 0).astype(jnp.float32)
    dy = y - my
    gyt = jnp.exp(-(dy * dy) * hv_r)

    x = jax.lax.broadcasted_iota(jnp.int32, (np_, ws), 1).astype(jnp.float32)
    dx = x - mx
    cgx = ceff * jnp.exp(-(dx * dx) * hv_c)

    out_ref[0] = jnp.dot(gyt, cgx, preferred_element_type=jnp.float32)


def kernel(mean, variance, confidence):
    B, H, W, _ = mean.shape
    Hs, Ws = H * _STRIDE, W * _STRIDE
    N = H * W
    NP = ((N + 127) // 128) * 128 if N > 224 else 224
    pad = NP - N

    m = mean.reshape(B, N, 2)
    mx = jnp.pad(m[..., 0], ((0, 0), (0, pad)))
    my = jnp.pad(m[..., 1], ((0, 0), (0, pad)))
    v = jnp.pad(variance.reshape(B, N), ((0, 0), (0, pad)), constant_values=1.0)
    c = jnp.pad(confidence.reshape(B, N), ((0, 0), (0, pad)))

    my_row = my[:, None, :]
    v_row = v[:, None, :]
    mx_col = mx[:, :, None]
    v_col = v[:, :, None]
    c_col = c[:, :, None]

    row_spec = pl.BlockSpec((1, 1, NP), lambda b: (b, 0, 0))
    col_spec = pl.BlockSpec((1, NP, 1), lambda b: (b, 0, 0))
    return pl.pallas_call(
        _body,
        grid=(B,),
        in_specs=[row_spec, row_spec, col_spec, col_spec, col_spec],
        out_specs=pl.BlockSpec((1, Hs, Ws), lambda b: (b, 0, 0)),
        out_shape=jax.ShapeDtypeStruct((B, Hs, Ws), jnp.float32),
    )(my_row, v_row, mx_col, v_col, c_col)
